# baseline (device time: 214455 ns/iter reference)
import functools

import jax
import jax.numpy as jnp
from jax import lax
from jax.experimental import pallas as pl
from jax.experimental.pallas import tpu as pltpu

N_DEV = 4
SQ = 1024
SKV = 1024
HQ = 8
DH = 128
D = HQ * DH
BLK = 64
SCALE = 0.08838834764831843


def kernel(x, Wq, K_ext, V_ext, Wo):
    def body(x_ref, wq_ref, k_ref, v_ref, wo_ref, out_ref,
             kv_buf, send_sems, recv_sem):
        my = lax.axis_index("i")
        left = (my - 1) % N_DEV
        right = (my + 1) % N_DEV

        barrier_sem = pltpu.get_barrier_semaphore()
        for nbr in (left, right):
            pl.semaphore_signal(
                barrier_sem, inc=1,
                device_id=(nbr,), device_id_type=pl.DeviceIdType.MESH,
            )
        pl.semaphore_wait(barrier_sem, 2)

        send1 = pltpu.make_async_remote_copy(
            src_ref=kv_buf, dst_ref=kv_buf,
            send_sem=send_sems.at[0], recv_sem=recv_sem,
            device_id=(1,), device_id_type=pl.DeviceIdType.MESH,
        )
        send3 = pltpu.make_async_remote_copy(
            src_ref=kv_buf, dst_ref=kv_buf,
            send_sem=send_sems.at[1], recv_sem=recv_sem,
            device_id=(3,), device_id_type=pl.DeviceIdType.MESH,
        )
        fwd2 = pltpu.make_async_remote_copy(
            src_ref=kv_buf, dst_ref=kv_buf,
            send_sem=send_sems.at[0], recv_sem=recv_sem,
            device_id=(2,), device_id_type=pl.DeviceIdType.MESH,
        )

        @pl.when(my == 0)
        def _():
            kv_buf[0] = k_ref[0].reshape(SKV, D)
            kv_buf[1] = v_ref[0].reshape(SKV, D)
            send1.start()
            send3.start()

        q = jnp.dot(x_ref[0], wq_ref[...], preferred_element_type=jnp.float32)

        @pl.when(my != 0)
        def _():
            send1.wait_recv()

        @pl.when(my == 1)
        def _():
            fwd2.start()

        row = lax.broadcasted_iota(jnp.int32, (SQ, SKV), 0) // BLK
        col = lax.broadcasted_iota(jnp.int32, (SQ, SKV), 1) // BLK
        mask = col <= row

        kbuf = kv_buf[0]
        vbuf = kv_buf[1]
        ctx_parts = []
        for h in range(HQ):
            qh = q[:, h * DH:(h + 1) * DH]
            kh = kbuf[:, h * DH:(h + 1) * DH]
            vh = vbuf[:, h * DH:(h + 1) * DH]
            s = lax.dot_general(
                qh, kh, (((1,), (1,)), ((), ())),
                preferred_element_type=jnp.float32,
            ) * SCALE
            s = jnp.where(mask, s, -1e9)
            m = jnp.max(s, axis=-1, keepdims=True)
            w = jnp.exp(s - m)
            w = w / jnp.sum(w, axis=-1, keepdims=True)
            ctx_parts.append(
                jnp.dot(w, vh, preferred_element_type=jnp.float32)
            )
        ctx = jnp.concatenate(ctx_parts, axis=-1)
        out_ref[0] = jnp.dot(ctx, wo_ref[...],
                             preferred_element_type=jnp.float32)

        @pl.when(my == 0)
        def _():
            send1.wait_send()
            send3.wait_send()

        @pl.when(my == 1)
        def _():
            fwd2.wait_send()

        @functools.partial(pl.run_scoped,
                           second_barrier=pltpu.SemaphoreType.REGULAR)
        def _(second_barrier):
            for nbr in (left, right):
                pl.semaphore_signal(
                    second_barrier, inc=1,
                    device_id=(nbr,), device_id_type=pl.DeviceIdType.MESH,
                )
            pl.semaphore_wait(second_barrier, 2)

    return pl.pallas_call(
        body,
        out_shape=jax.ShapeDtypeStruct((1, SQ, D), jnp.float32),
        in_specs=[pl.BlockSpec(memory_space=pltpu.VMEM)] * 5,
        out_specs=pl.BlockSpec(memory_space=pltpu.VMEM),
        scratch_shapes=[
            pltpu.VMEM((2, SKV, D), jnp.float32),
            pltpu.SemaphoreType.DMA((2,)),
            pltpu.SemaphoreType.DMA,
        ],
        compiler_params=pltpu.CompilerParams(collective_id=0),
    )(x, Wq, K_ext, V_ext, Wo)


# device time: 74487 ns/iter; 2.8791x vs baseline; 2.8791x over previous
import functools

import jax
import jax.numpy as jnp
from jax import lax
from jax.experimental import pallas as pl
from jax.experimental.pallas import tpu as pltpu

N_DEV = 4
SQ = 1024
SKV = 1024
HQ = 8
DH = 128
D = HQ * DH
BLK = 64
SCALE = 0.08838834764831843

NC = 8
CR = SQ // NC


def kernel(x, Wq, K_ext, V_ext, Wo):
    def body(x_ref, wq_ref, k_ref, v_ref, wo_ref, out_ref,
             kvflat, sem1, sem3, recv_sems):
        my = lax.axis_index("i")
        left = (my - 1) % N_DEV
        right = (my + 1) % N_DEV

        barrier_sem = pltpu.get_barrier_semaphore()
        for nbr in (left, right):
            pl.semaphore_signal(
                barrier_sem, inc=1,
                device_id=(nbr,), device_id_type=pl.DeviceIdType.MESH,
            )
        pl.semaphore_wait(barrier_sem, 2)

        def chunk_ref(c):
            return out_ref.at[0, pl.ds(c * CR, CR), :]

        @pl.when(my == 0)
        def _():
            kvflat[0] = k_ref[0].reshape(SKV, D)
            kvflat[1] = v_ref[0].reshape(SKV, D)
            kbuf = kvflat[0]
            vbuf = kvflat[1]
            rdmas = []
            for c in range(NC):
                rows = pl.ds(c * CR, CR)
                L = CR * (c + 1)
                qc = jnp.dot(x_ref[0, rows, :], wq_ref[...],
                             preferred_element_type=jnp.float32)
                rowb = (c * CR
                        + lax.broadcasted_iota(jnp.int32, (CR, L), 0)) // BLK
                colb = lax.broadcasted_iota(jnp.int32, (CR, L), 1) // BLK
                mask = colb <= rowb
                ctx_parts = []
                for h in range(HQ):
                    qh = qc[:, h * DH:(h + 1) * DH]
                    kh = kbuf[0:L, h * DH:(h + 1) * DH]
                    vh = vbuf[0:L, h * DH:(h + 1) * DH]
                    s = lax.dot_general(
                        qh, kh, (((1,), (1,)), ((), ())),
                        preferred_element_type=jnp.float32,
                    ) * SCALE
                    s = jnp.where(mask, s, -1e9)
                    m = jnp.max(s, axis=-1, keepdims=True)
                    w = jnp.exp(s - m)
                    w = w / jnp.sum(w, axis=-1, keepdims=True)
                    ctx_parts.append(
                        jnp.dot(w, vh, preferred_element_type=jnp.float32)
                    )
                ctx = jnp.concatenate(ctx_parts, axis=-1)
                out_ref[0, rows, :] = jnp.dot(
                    ctx, wo_ref[...], preferred_element_type=jnp.float32)
                for tgt, sems in ((1, sem1), (3, sem3)):
                    r = pltpu.make_async_remote_copy(
                        src_ref=chunk_ref(c), dst_ref=chunk_ref(c),
                        send_sem=sems.at[c], recv_sem=recv_sems.at[c],
                        device_id=(tgt,),
                        device_id_type=pl.DeviceIdType.MESH,
                    )
                    r.start()
                    rdmas.append(r)
            for r in rdmas:
                r.wait_send()

        @pl.when(my == 1)
        def _():
            fwds = []
            for c in range(NC):
                rc = pltpu.make_async_remote_copy(
                    src_ref=chunk_ref(c), dst_ref=chunk_ref(c),
                    send_sem=sem1.at[c], recv_sem=recv_sems.at[c],
                    device_id=(2,), device_id_type=pl.DeviceIdType.MESH,
                )
                rc.wait_recv()
                rc.start()
                fwds.append(rc)
            for r in fwds:
                r.wait_send()

        @pl.when((my == 2) | (my == 3))
        def _():
            for c in range(NC):
                rc = pltpu.make_async_remote_copy(
                    src_ref=chunk_ref(c), dst_ref=chunk_ref(c),
                    send_sem=sem1.at[c], recv_sem=recv_sems.at[c],
                    device_id=(0,), device_id_type=pl.DeviceIdType.MESH,
                )
                rc.wait_recv()

        @functools.partial(pl.run_scoped,
                           second_barrier=pltpu.SemaphoreType.REGULAR)
        def _(second_barrier):
            for nbr in (left, right):
                pl.semaphore_signal(
                    second_barrier, inc=1,
                    device_id=(nbr,), device_id_type=pl.DeviceIdType.MESH,
                )
            pl.semaphore_wait(second_barrier, 2)

    return pl.pallas_call(
        body,
        out_shape=jax.ShapeDtypeStruct((1, SQ, D), jnp.float32),
        in_specs=[pl.BlockSpec(memory_space=pltpu.VMEM)] * 5,
        out_specs=pl.BlockSpec(memory_space=pltpu.VMEM),
        scratch_shapes=[
            pltpu.VMEM((2, SKV, D), jnp.float32),
            pltpu.SemaphoreType.DMA((NC,)),
            pltpu.SemaphoreType.DMA((NC,)),
            pltpu.SemaphoreType.DMA((NC,)),
        ],
        compiler_params=pltpu.CompilerParams(collective_id=0),
    )(x, Wq, K_ext, V_ext, Wo)


# device time: 53607 ns/iter; 4.0005x vs baseline; 1.3895x over previous
import functools

import jax
import jax.numpy as jnp
from jax import lax
from jax.experimental import pallas as pl
from jax.experimental.pallas import tpu as pltpu

N_DEV = 4
SQ = 1024
SKV = 1024
HQ = 8
DH = 128
D = HQ * DH
BLK = 64
SCALE = 0.08838834764831843

NC = 8
CR = SQ // NC


def kernel(x, Wq, K_ext, V_ext, Wo):
    def body(x_ref, wq_ref, k_ref, v_ref, wo_ref, out_ref,
             kvflat, cbuf, sem1, sem3, recv_sems):
        my = lax.axis_index("i")
        left = (my - 1) % N_DEV
        right = (my + 1) % N_DEV

        barrier_sem = pltpu.get_barrier_semaphore()
        for nbr in (left, right):
            pl.semaphore_signal(
                barrier_sem, inc=1,
                device_id=(nbr,), device_id_type=pl.DeviceIdType.MESH,
            )
        pl.semaphore_wait(barrier_sem, 2)

        def chunk_ref(c):
            return cbuf.at[pl.ds(c * CR, CR), :]

        def project_chunk(c):
            rows = pl.ds(c * CR, CR)
            ctxc = cbuf[rows, :].astype(jnp.float32)
            out_ref[0, rows, :] = jnp.dot(
                ctxc, wo_ref[...], preferred_element_type=jnp.float32)

        @pl.when(my == 0)
        def _():
            kvflat[0] = k_ref[0].reshape(SKV, D)
            kvflat[1] = v_ref[0].reshape(SKV, D)
            kbuf = kvflat[0]
            vbuf = kvflat[1]
            rdmas = []
            for c in range(NC):
                rows = pl.ds(c * CR, CR)
                L = CR * (c + 1)
                qc = jnp.dot(x_ref[0, rows, :], wq_ref[...],
                             preferred_element_type=jnp.float32)
                rowb = (c * CR
                        + lax.broadcasted_iota(jnp.int32, (CR, L), 0)) // BLK
                colb = lax.broadcasted_iota(jnp.int32, (CR, L), 1) // BLK
                mask = colb <= rowb
                ctx_parts = []
                for h in range(HQ):
                    qh = qc[:, h * DH:(h + 1) * DH]
                    kh = kbuf[0:L, h * DH:(h + 1) * DH]
                    vh = vbuf[0:L, h * DH:(h + 1) * DH]
                    s = lax.dot_general(
                        qh, kh, (((1,), (1,)), ((), ())),
                        preferred_element_type=jnp.float32,
                    ) * SCALE
                    s = jnp.where(mask, s, -1e9)
                    m = jnp.max(s, axis=-1, keepdims=True)
                    w = jnp.exp(s - m)
                    w = w / jnp.sum(w, axis=-1, keepdims=True)
                    ctx_parts.append(
                        jnp.dot(w, vh, preferred_element_type=jnp.float32)
                    )
                ctx = jnp.concatenate(ctx_parts, axis=-1)
                cbuf[rows, :] = ctx.astype(jnp.bfloat16)
                for tgt, sems in ((1, sem1), (3, sem3)):
                    r = pltpu.make_async_remote_copy(
                        src_ref=chunk_ref(c), dst_ref=chunk_ref(c),
                        send_sem=sems.at[c], recv_sem=recv_sems.at[c],
                        device_id=(tgt,),
                        device_id_type=pl.DeviceIdType.MESH,
                    )
                    r.start()
                    rdmas.append(r)
            for c in range(NC):
                project_chunk(c)
            for r in rdmas:
                r.wait_send()

        @pl.when(my == 1)
        def _():
            fwds = []
            for c in range(NC):
                rc = pltpu.make_async_remote_copy(
                    src_ref=chunk_ref(c), dst_ref=chunk_ref(c),
                    send_sem=sem1.at[c], recv_sem=recv_sems.at[c],
                    device_id=(2,), device_id_type=pl.DeviceIdType.MESH,
                )
                rc.wait_recv()
                rc.start()
                fwds.append(rc)
                project_chunk(c)
            for r in fwds:
                r.wait_send()

        @pl.when((my == 2) | (my == 3))
        def _():
            for c in range(NC):
                rc = pltpu.make_async_remote_copy(
                    src_ref=chunk_ref(c), dst_ref=chunk_ref(c),
                    send_sem=sem1.at[c], recv_sem=recv_sems.at[c],
                    device_id=(0,), device_id_type=pl.DeviceIdType.MESH,
                )
                rc.wait_recv()
                project_chunk(c)

        @functools.partial(pl.run_scoped,
                           second_barrier=pltpu.SemaphoreType.REGULAR)
        def _(second_barrier):
            for nbr in (left, right):
                pl.semaphore_signal(
                    second_barrier, inc=1,
                    device_id=(nbr,), device_id_type=pl.DeviceIdType.MESH,
                )
            pl.semaphore_wait(second_barrier, 2)

    return pl.pallas_call(
        body,
        out_shape=jax.ShapeDtypeStruct((1, SQ, D), jnp.float32),
        in_specs=[pl.BlockSpec(memory_space=pltpu.VMEM)] * 5,
        out_specs=pl.BlockSpec(memory_space=pltpu.VMEM),
        scratch_shapes=[
            pltpu.VMEM((2, SKV, D), jnp.float32),
            pltpu.VMEM((SQ, D), jnp.bfloat16),
            pltpu.SemaphoreType.DMA((NC,)),
            pltpu.SemaphoreType.DMA((NC,)),
            pltpu.SemaphoreType.DMA((NC,)),
        ],
        compiler_params=pltpu.CompilerParams(collective_id=0),
    )(x, Wq, K_ext, V_ext, Wo)


# device time: 42772 ns/iter; 5.0139x vs baseline; 1.2533x over previous
import functools

import jax
import jax.numpy as jnp
from jax import lax
from jax.experimental import pallas as pl
from jax.experimental.pallas import tpu as pltpu

N_DEV = 4
SQ = 1024
SKV = 1024
HQ = 8
DH = 128
D = HQ * DH
BLK = 64
SCALE = 0.08838834764831843

NC = 8
CR = SQ // NC


def kernel(x, Wq, K_ext, V_ext, Wo):
    def body(x_ref, wq_ref, k_ref, v_ref, wo_ref, out_ref,
             kvflat, cbuf, sem1, sem3, recv_sems):
        my = lax.axis_index("i")
        left = (my - 1) % N_DEV
        right = (my + 1) % N_DEV

        barrier_sem = pltpu.get_barrier_semaphore()
        for nbr in (left, right):
            pl.semaphore_signal(
                barrier_sem, inc=1,
                device_id=(nbr,), device_id_type=pl.DeviceIdType.MESH,
            )
        pl.semaphore_wait(barrier_sem, 2)

        def chunk_ref(c):
            return cbuf.at[pl.ds(c * CR, CR), :]

        def project_chunk(c):
            rows = pl.ds(c * CR, CR)
            ctxc = cbuf[rows, :].astype(jnp.float32)
            out_ref[0, rows, :] = jnp.dot(
                ctxc, wo_ref[...], preferred_element_type=jnp.float32)

        @pl.when(my == 0)
        def _():
            kvflat[0] = k_ref[0].reshape(SKV, D).astype(jnp.bfloat16)
            kvflat[1] = v_ref[0].reshape(SKV, D).astype(jnp.bfloat16)
            kbuf = kvflat[0]
            vbuf = kvflat[1]
            q = jnp.dot(x_ref[0], wq_ref[...],
                        preferred_element_type=jnp.float32
                        ).astype(jnp.bfloat16)
            rdmas = []
            for c in range(NC):
                rows = pl.ds(c * CR, CR)
                L = CR * (c + 1)
                rowb = (c * CR
                        + lax.broadcasted_iota(jnp.int32, (CR, L), 0)) // BLK
                colb = lax.broadcasted_iota(jnp.int32, (CR, L), 1) // BLK
                mask = colb <= rowb
                ctx_parts = []
                for h in range(HQ):
                    qh = q[c * CR:(c + 1) * CR, h * DH:(h + 1) * DH]
                    kh = kbuf[0:L, h * DH:(h + 1) * DH]
                    vh = vbuf[0:L, h * DH:(h + 1) * DH]
                    s = lax.dot_general(
                        qh, kh, (((1,), (1,)), ((), ())),
                        preferred_element_type=jnp.float32,
                    ) * SCALE
                    s = jnp.where(mask, s, -1e9)
                    m = jnp.max(s, axis=-1, keepdims=True)
                    w = jnp.exp(s - m)
                    w = (w / jnp.sum(w, axis=-1, keepdims=True)
                         ).astype(jnp.bfloat16)
                    ctx_parts.append(
                        jnp.dot(w, vh, preferred_element_type=jnp.float32)
                    )
                ctx = jnp.concatenate(ctx_parts, axis=-1)
                cbuf[rows, :] = ctx.astype(jnp.bfloat16)
                for tgt, sems in ((1, sem1), (3, sem3)):
                    r = pltpu.make_async_remote_copy(
                        src_ref=chunk_ref(c), dst_ref=chunk_ref(c),
                        send_sem=sems.at[c], recv_sem=recv_sems.at[c],
                        device_id=(tgt,),
                        device_id_type=pl.DeviceIdType.MESH,
                    )
                    r.start()
                    rdmas.append(r)
            for c in range(NC):
                project_chunk(c)
            for r in rdmas:
                r.wait_send()

        @pl.when(my == 1)
        def _():
            fwds = []
            for c in range(NC):
                rc = pltpu.make_async_remote_copy(
                    src_ref=chunk_ref(c), dst_ref=chunk_ref(c),
                    send_sem=sem1.at[c], recv_sem=recv_sems.at[c],
                    device_id=(2,), device_id_type=pl.DeviceIdType.MESH,
                )
                rc.wait_recv()
                rc.start()
                fwds.append(rc)
                project_chunk(c)
            for r in fwds:
                r.wait_send()

        @pl.when((my == 2) | (my == 3))
        def _():
            for c in range(NC):
                rc = pltpu.make_async_remote_copy(
                    src_ref=chunk_ref(c), dst_ref=chunk_ref(c),
                    send_sem=sem1.at[c], recv_sem=recv_sems.at[c],
                    device_id=(0,), device_id_type=pl.DeviceIdType.MESH,
                )
                rc.wait_recv()
                project_chunk(c)

        @functools.partial(pl.run_scoped,
                           second_barrier=pltpu.SemaphoreType.REGULAR)
        def _(second_barrier):
            for nbr in (left, right):
                pl.semaphore_signal(
                    second_barrier, inc=1,
                    device_id=(nbr,), device_id_type=pl.DeviceIdType.MESH,
                )
            pl.semaphore_wait(second_barrier, 2)

    return pl.pallas_call(
        body,
        out_shape=jax.ShapeDtypeStruct((1, SQ, D), jnp.float32),
        in_specs=[pl.BlockSpec(memory_space=pltpu.VMEM)] * 5,
        out_specs=pl.BlockSpec(memory_space=pltpu.VMEM),
        scratch_shapes=[
            pltpu.VMEM((2, SKV, D), jnp.bfloat16),
            pltpu.VMEM((SQ, D), jnp.bfloat16),
            pltpu.SemaphoreType.DMA((NC,)),
            pltpu.SemaphoreType.DMA((NC,)),
            pltpu.SemaphoreType.DMA((NC,)),
        ],
        compiler_params=pltpu.CompilerParams(collective_id=0),
    )(x, Wq, K_ext, V_ext, Wo)
